# baseline (speedup 1.0000x reference)
import functools

import jax
import jax.numpy as jnp
from jax import lax
from jax.experimental import pallas as pl
from jax.experimental.pallas import tpu as pltpu

_B, _D, _E, _K = 32768, 4096, 64, 8
_BT = 1024
_NB = _B // _BT


def _router_body(x_ref, w_ref, b_ref, gw_ref, tki_ref, tkw_ref, lb_ref,
                 sp_ref, lt_ref, imp_ref, cnt_ref, tks_ref):
    i = pl.program_id(0)

    logits = jnp.dot(x_ref[...], w_ref[...],
                     preferred_element_type=jnp.float32) + b_ref[...]
    lt_ref[i % 2] = logits.T

    lt = lt_ref[(i + 1) % 2]
    m = jnp.max(lt, axis=0, keepdims=True)
    e = jnp.exp(lt - m)
    s = jnp.sum(e, axis=0, keepdims=True)
    gwt = e * (1.0 / s)
    gw_ref[...] = gwt.T

    iota = lax.broadcasted_iota(jnp.int32, (_E, _BT), 0)
    g = gwt
    w_rows = []
    i_rows = []
    for _ in range(_K):
        mx = jnp.max(g, axis=0, keepdims=True)
        idx = jnp.min(jnp.where(g == mx, iota, _E), axis=0, keepdims=True)
        w_rows.append(mx)
        i_rows.append(idx)
        g = jnp.where(iota == idx, -1.0, g)
    tkw_t = jnp.concatenate(w_rows, axis=0)
    tki_t = jnp.concatenate(i_rows, axis=0)
    tkw_ref[...] = tkw_t.T
    tki_ref[...] = tki_t.T

    sel = (g < 0.0).astype(jnp.float32)
    tks_blk = jnp.sum(tkw_t, axis=0, keepdims=True)

    fresh = i <= 1
    imp_ref[...] = jnp.where(fresh, 0.0, imp_ref[...]) + gwt
    cnt_ref[...] = jnp.where(fresh, 0.0, cnt_ref[...]) + sel
    tks_ref[...] = jnp.where(fresh, 0.0, tks_ref[...]) + tks_blk

    @pl.when(i == _NB)
    def _finalize():
        imp_col = jnp.sum(imp_ref[...], axis=1, keepdims=True)
        cnt_col = jnp.sum(cnt_ref[...], axis=1, keepdims=True)
        lb_ref[0, 0] = (_E / (_B * float(_B))) * jnp.sum(imp_col * cnt_col)
        sp_ref[0, 0] = 1.0 - jnp.sum(tks_ref[...]) * (1.0 / _B)


@functools.partial(jax.jit, static_argnames=())
def kernel(x, W, b):
    b2 = b.reshape(1, _E)
    out_shape = (
        jax.ShapeDtypeStruct((_B, _E), jnp.float32),
        jax.ShapeDtypeStruct((_B, _K), jnp.int32),
        jax.ShapeDtypeStruct((_B, _K), jnp.float32),
        jax.ShapeDtypeStruct((1, 1), jnp.float32),
        jax.ShapeDtypeStruct((1, 1), jnp.float32),
    )
    grid = (_NB + 1,)
    prev = lambda i: (jnp.maximum(i - 1, 0), 0)
    gw, tki, tkw, lb, sp = pl.pallas_call(
        _router_body,
        grid=grid,
        in_specs=[
            pl.BlockSpec((_BT, _D), lambda i: (jnp.minimum(i, _NB - 1), 0)),
            pl.BlockSpec((_D, _E), lambda i: (0, 0)),
            pl.BlockSpec((1, _E), lambda i: (0, 0)),
        ],
        out_specs=(
            pl.BlockSpec((_BT, _E), prev),
            pl.BlockSpec((_BT, _K), prev),
            pl.BlockSpec((_BT, _K), prev),
            pl.BlockSpec(memory_space=pltpu.SMEM),
            pl.BlockSpec(memory_space=pltpu.SMEM),
        ),
        out_shape=out_shape,
        scratch_shapes=[
            pltpu.VMEM((2, _E, _BT), jnp.float32),
            pltpu.VMEM((_E, _BT), jnp.float32),
            pltpu.VMEM((_E, _BT), jnp.float32),
            pltpu.VMEM((1, _BT), jnp.float32),
        ],
    )(x, W, b2)
    return (gw, tki, tkw, lb.reshape(()), sp.reshape(()))
